# baseline (speedup 1.0000x reference)
import jax
import jax.numpy as jnp
from jax import lax
from jax.experimental import pallas as pl
from jax.experimental.pallas import tpu as pltpu
from jax.experimental.pallas import tpu_sc as plsc

N = 10000
E = 320000
D = 128

NC = 2
NS = 16
NW = NC * NS

CH = 80
CPW = 128
PAD_ROWS = NW * CPW
E_PAD = PAD_ROWS * CH

IGRP = 32
NGRP = CPW // IGRP
NBUF = 4

N_PAD = 10112
RPT = N_PAD // NS

DEGW = 128

_mesh = plsc.VectorSubcoreMesh(
    core_axis_name="c", subcore_axis_name="s", num_cores=NC, num_subcores=NS)


def _deg_body(dstc_hbm, zrows_hbm, ones_hbm, out_hbm, didx, ones_v, dacc):
    c = lax.axis_index("c")
    s = lax.axis_index("s")
    w = s * NC + c
    base = w * CPW

    pltpu.sync_copy(dstc_hbm.at[pl.ds(base, CPW)], didx)
    pltpu.sync_copy(ones_hbm, ones_v)
    pltpu.sync_copy(zrows_hbm, dacc.at[pl.ds(s * RPT, RPT)])
    plsc.subcore_barrier()

    def body(j, _):
        pltpu.sync_copy(ones_v, dacc.at[didx.at[j]], add=True)
        return 0
    lax.fori_loop(0, CPW, body, 0)

    plsc.subcore_barrier()
    pltpu.sync_copy(dacc.at[pl.ds(s * RPT, RPT)],
                    out_hbm.at[c, pl.ds(s * RPT, RPT)])


_deg_kernel = pl.kernel(
    _deg_body,
    out_type=jax.ShapeDtypeStruct((NC, N_PAD, DEGW), jnp.float32),
    mesh=_mesh,
    scratch_types=[
        pltpu.VMEM((CPW, CH), jnp.int32),
        pltpu.VMEM((CH, DEGW), jnp.float32),
        pltpu.VMEM_SHARED((N_PAD, DEGW), jnp.float32),
    ],
)


def _mp_body(srcc_hbm, dstc_hbm, g_hbm, zrows_hbm, out_hbm,
             sidx, didx, rows, acc, sems):
    c = lax.axis_index("c")
    s = lax.axis_index("s")
    w = s * NC + c
    base = w * CPW

    pltpu.sync_copy(zrows_hbm, acc.at[pl.ds(s * RPT, RPT)])
    plsc.subcore_barrier()

    def issue(j, b):
        pltpu.async_copy(g_hbm.at[sidx.at[j]], rows.at[b], sems.at[b])

    def group(g, _):
        gbase = base + g * IGRP
        pltpu.sync_copy(srcc_hbm.at[pl.ds(gbase, IGRP)], sidx)
        pltpu.sync_copy(dstc_hbm.at[pl.ds(gbase, IGRP)], didx)

        for b in range(NBUF - 1):
            issue(b, b)

        def body(i, _):
            j0 = i * NBUF
            for b in range(NBUF):
                j = j0 + b
                nb = (b + NBUF - 1) % NBUF
                @pl.when(j + NBUF - 1 < IGRP)
                def _():
                    issue(j + NBUF - 1, nb)
                pltpu.make_async_copy(
                    g_hbm.at[sidx.at[j]], rows.at[b], sems.at[b]).wait()
                pltpu.sync_copy(rows.at[b], acc.at[didx.at[j]], add=True)
            return 0
        lax.fori_loop(0, IGRP // NBUF, body, 0)
        return 0
    lax.fori_loop(0, NGRP, group, 0)

    plsc.subcore_barrier()
    pltpu.sync_copy(acc.at[pl.ds(s * RPT, RPT)],
                    out_hbm.at[c, pl.ds(s * RPT, RPT)])


_mp_kernel = pl.kernel(
    _mp_body,
    out_type=jax.ShapeDtypeStruct((NC, N_PAD, D), jnp.float32),
    mesh=_mesh,
    scratch_types=[
        pltpu.VMEM((IGRP, CH), jnp.int32),
        pltpu.VMEM((IGRP, CH), jnp.int32),
        pltpu.VMEM((NBUF, CH, D), jnp.float32),
        pltpu.VMEM_SHARED((N_PAD, D), jnp.float32),
        pltpu.SemaphoreType.DMA((NBUF,)),
    ],
)


BN = 1000


def _dinv_of(deg0_ref, deg1_ref):
    deg = deg0_ref[:, 0:1] + deg1_ref[:, 0:1] + 1.0
    return lax.rsqrt(jnp.maximum(deg, 1.0))


def _tc1_body(x_ref, deg0_ref, deg1_ref, w1_ref, g1_ref):
    dinv = _dinv_of(deg0_ref, deg1_ref)
    h = jnp.dot(x_ref[...], w1_ref[...], preferred_element_type=jnp.float32)
    g1_ref[...] = h * dinv


def _tc2_body(a0_ref, a1_ref, g1_ref, deg0_ref, deg1_ref,
              b1_ref, al1_ref, w2_ref, g2_ref):
    dinv = _dinv_of(deg0_ref, deg1_ref)
    pre = dinv * (a0_ref[...] + a1_ref[...] + g1_ref[...]) + b1_ref[...]
    z = jnp.where(pre >= 0, pre, al1_ref[...] * pre)
    h2 = jnp.dot(z, w2_ref[...], preferred_element_type=jnp.float32)
    g2_ref[...] = h2 * dinv


def _tc3_body(a0_ref, a1_ref, g2_ref, deg0_ref, deg1_ref,
              b2_ref, al2_ref, out_ref):
    dinv = _dinv_of(deg0_ref, deg1_ref)
    pre = dinv * (a0_ref[...] + a1_ref[...] + g2_ref[...]) + b2_ref[...]
    out_ref[...] = jnp.where(pre >= 0, pre, al2_ref[...] * pre)


def _row_spec(width):
    return pl.BlockSpec((BN, width), lambda i: (i, 0))


def _full_spec(shape):
    return pl.BlockSpec(shape, lambda i: tuple(0 for _ in shape))


def kernel(x, edge_index, W1, b1, a1, W2, b2, a2):
    src = edge_index[0]
    dst = edge_index[1]
    pad = E_PAD - E
    srcc = jnp.pad(src, (0, pad)).reshape(PAD_ROWS, CH)
    dstc = jnp.pad(dst, (0, pad), constant_values=N).reshape(PAD_ROWS, CH)

    zdeg = jnp.zeros((RPT, DEGW), jnp.float32)
    zacc = jnp.zeros((RPT, D), jnp.float32)

    ones = jnp.ones((CH, DEGW), jnp.float32)
    degp = _deg_kernel(dstc, zdeg, ones)
    deg0, deg1 = degp[0], degp[1]

    b1r = b1.reshape(1, D)
    a1r = a1.reshape(1, D)
    b2r = b2.reshape(1, D)
    a2r = a2.reshape(1, D)

    grid = (N // BN,)
    g1 = pl.pallas_call(
        _tc1_body,
        grid=grid,
        in_specs=[_row_spec(D), _row_spec(DEGW), _row_spec(DEGW),
                  _full_spec((D, D))],
        out_specs=_row_spec(D),
        out_shape=jax.ShapeDtypeStruct((N, D), jnp.float32),
    )(x, deg0, deg1, W1)

    acc1 = _mp_kernel(srcc, dstc, g1, zacc)

    g2 = pl.pallas_call(
        _tc2_body,
        grid=grid,
        in_specs=[_row_spec(D), _row_spec(D), _row_spec(D),
                  _row_spec(DEGW), _row_spec(DEGW),
                  _full_spec((1, D)), _full_spec((1, D)), _full_spec((D, D))],
        out_specs=_row_spec(D),
        out_shape=jax.ShapeDtypeStruct((N, D), jnp.float32),
    )(acc1[0], acc1[1], g1, deg0, deg1, b1r, a1r, W2)

    acc2 = _mp_kernel(srcc, dstc, g2, zacc)

    out = pl.pallas_call(
        _tc3_body,
        grid=grid,
        in_specs=[_row_spec(D), _row_spec(D), _row_spec(D),
                  _row_spec(DEGW), _row_spec(DEGW),
                  _full_spec((1, D)), _full_spec((1, D))],
        out_specs=_row_spec(D),
        out_shape=jax.ShapeDtypeStruct((N, D), jnp.float32),
    )(acc2[0], acc2[1], g2, deg0, deg1, b2r, a2r)

    return out


# speedup vs baseline: 3.2478x; 3.2478x over previous
import jax
import jax.numpy as jnp
from jax import lax
from jax.experimental import pallas as pl
from jax.experimental.pallas import tpu as pltpu
from jax.experimental.pallas import tpu_sc as plsc

N = 10000
E = 320000
D = 128

NC = 2
NS = 16
NW = NC * NS

CH = 80
CPW = 128
PAD_ROWS = NW * CPW
E_PAD = PAD_ROWS * CH

IGRP = 32
NGRP = CPW // IGRP
NBUF = 4

N_PAD = 10112
RPT = N_PAD // NS

DEGW = 128

_mesh = plsc.VectorSubcoreMesh(
    core_axis_name="c", subcore_axis_name="s", num_cores=NC, num_subcores=NS)


def _deg_body(dstc_hbm, zrows_hbm, ones_hbm, out_hbm, didx, ones_v, dacc):
    c = lax.axis_index("c")
    s = lax.axis_index("s")
    w = s * NC + c
    base = w * CPW

    pltpu.sync_copy(dstc_hbm.at[pl.ds(base, CPW)], didx)
    pltpu.sync_copy(ones_hbm, ones_v)
    pltpu.sync_copy(zrows_hbm, dacc.at[pl.ds(s * RPT, RPT)])
    plsc.subcore_barrier()

    def body(j, _):
        pltpu.sync_copy(ones_v, dacc.at[didx.at[j]], add=True)
        return 0
    lax.fori_loop(0, CPW, body, 0)

    plsc.subcore_barrier()
    pltpu.sync_copy(dacc.at[pl.ds(s * RPT, RPT)],
                    out_hbm.at[c, pl.ds(s * RPT, RPT)])


_deg_kernel = pl.kernel(
    _deg_body,
    out_type=jax.ShapeDtypeStruct((NC, N_PAD, DEGW), jnp.float32),
    mesh=_mesh,
    scratch_types=[
        pltpu.VMEM((CPW, CH), jnp.int32),
        pltpu.VMEM((CH, DEGW), jnp.float32),
        pltpu.VMEM_SHARED((N_PAD, DEGW), jnp.float32),
    ],
)


def _mp_body(srcc_hbm, dstc_hbm, g_hbm, zrows_hbm, out_hbm,
             sidx, didx, rows, acc, sems):
    c = lax.axis_index("c")
    s = lax.axis_index("s")
    w = s * NC + c
    base = w * CPW

    pltpu.sync_copy(zrows_hbm, acc.at[pl.ds(s * RPT, RPT)])
    plsc.subcore_barrier()

    def issue(j, b):
        pltpu.async_copy(g_hbm.at[sidx.at[j]], rows.at[b], sems.at[b])

    def group(g, _):
        gbase = base + g * IGRP
        pltpu.sync_copy(srcc_hbm.at[pl.ds(gbase, IGRP)], sidx)
        pltpu.sync_copy(dstc_hbm.at[pl.ds(gbase, IGRP)], didx)

        for b in range(NBUF - 1):
            issue(b, b)

        def body(i, _):
            j0 = i * NBUF
            for b in range(NBUF):
                j = j0 + b
                nb = (b + NBUF - 1) % NBUF
                @pl.when(j + NBUF - 1 < IGRP)
                def _():
                    issue(j + NBUF - 1, nb)
                pltpu.make_async_copy(
                    g_hbm.at[sidx.at[j]], rows.at[b], sems.at[b]).wait()
                pltpu.sync_copy(rows.at[b], acc.at[didx.at[j]], add=True)
            return 0
        lax.fori_loop(0, IGRP // NBUF, body, 0)
        return 0
    lax.fori_loop(0, NGRP, group, 0)

    plsc.subcore_barrier()
    pltpu.sync_copy(acc.at[pl.ds(s * RPT, RPT)],
                    out_hbm.at[c, pl.ds(s * RPT, RPT)])


_mp_kernel = pl.kernel(
    _mp_body,
    out_type=jax.ShapeDtypeStruct((NC, N_PAD, D), jnp.float32),
    mesh=_mesh,
    scratch_types=[
        pltpu.VMEM((IGRP, CH), jnp.int32),
        pltpu.VMEM((IGRP, CH), jnp.int32),
        pltpu.VMEM((NBUF, CH, D), jnp.float32),
        pltpu.VMEM_SHARED((N_PAD, D), jnp.float32),
        pltpu.SemaphoreType.DMA((NBUF,)),
    ],
)


BN = 1000


def _dinv_of(deg0_ref, deg1_ref):
    deg = deg0_ref[:, 0:1] + deg1_ref[:, 0:1] + 1.0
    return lax.rsqrt(jnp.maximum(deg, 1.0))


def _tc1_body(x_ref, deg0_ref, deg1_ref, w1_ref, g1_ref):
    dinv = _dinv_of(deg0_ref, deg1_ref)
    h = jnp.dot(x_ref[...], w1_ref[...], preferred_element_type=jnp.float32)
    g1_ref[...] = h * dinv


def _tc2_body(a0_ref, a1_ref, g1_ref, deg0_ref, deg1_ref,
              b1_ref, al1_ref, w2_ref, g2_ref):
    dinv = _dinv_of(deg0_ref, deg1_ref)
    pre = dinv * (a0_ref[...] + a1_ref[...] + g1_ref[...]) + b1_ref[...]
    z = jnp.where(pre >= 0, pre, al1_ref[...] * pre)
    h2 = jnp.dot(z, w2_ref[...], preferred_element_type=jnp.float32)
    g2_ref[...] = h2 * dinv


def _tc3_body(a0_ref, a1_ref, g2_ref, deg0_ref, deg1_ref,
              b2_ref, al2_ref, out_ref):
    dinv = _dinv_of(deg0_ref, deg1_ref)
    pre = dinv * (a0_ref[...] + a1_ref[...] + g2_ref[...]) + b2_ref[...]
    out_ref[...] = jnp.where(pre >= 0, pre, al2_ref[...] * pre)


def _row_spec(width):
    return pl.BlockSpec((BN, width), lambda i: (i, 0))


def _full_spec(shape):
    return pl.BlockSpec(shape, lambda i: tuple(0 for _ in shape))


def kernel(x, edge_index, W1, b1, a1, W2, b2, a2):
    rcpw = E // (NW * CH)
    src3 = edge_index[0].reshape(NW, rcpw, CH)
    dst3 = edge_index[1].reshape(NW, rcpw, CH)
    npadc = CPW - rcpw
    pad_src = jnp.broadcast_to(
        jnp.arange(npadc * CH, dtype=jnp.int32).reshape(1, npadc, CH),
        (NW, npadc, CH))
    pad_dst = jnp.broadcast_to(
        (N + jnp.arange(npadc * CH, dtype=jnp.int32) % (N_PAD - N)
         ).reshape(1, npadc, CH),
        (NW, npadc, CH))
    srcc = jnp.concatenate([src3, pad_src], axis=1).reshape(PAD_ROWS, CH)
    dstc = jnp.concatenate([dst3, pad_dst], axis=1).reshape(PAD_ROWS, CH)

    zdeg = jnp.zeros((RPT, DEGW), jnp.float32)
    zacc = jnp.zeros((RPT, D), jnp.float32)

    ones = jnp.ones((CH, DEGW), jnp.float32)
    degp = _deg_kernel(dstc, zdeg, ones)
    deg0, deg1 = degp[0], degp[1]

    b1r = b1.reshape(1, D)
    a1r = a1.reshape(1, D)
    b2r = b2.reshape(1, D)
    a2r = a2.reshape(1, D)

    grid = (N // BN,)
    g1 = pl.pallas_call(
        _tc1_body,
        grid=grid,
        in_specs=[_row_spec(D), _row_spec(DEGW), _row_spec(DEGW),
                  _full_spec((D, D))],
        out_specs=_row_spec(D),
        out_shape=jax.ShapeDtypeStruct((N, D), jnp.float32),
    )(x, deg0, deg1, W1)

    acc1 = _mp_kernel(srcc, dstc, g1, zacc)

    g2 = pl.pallas_call(
        _tc2_body,
        grid=grid,
        in_specs=[_row_spec(D), _row_spec(D), _row_spec(D),
                  _row_spec(DEGW), _row_spec(DEGW),
                  _full_spec((1, D)), _full_spec((1, D)), _full_spec((D, D))],
        out_specs=_row_spec(D),
        out_shape=jax.ShapeDtypeStruct((N, D), jnp.float32),
    )(acc1[0], acc1[1], g1, deg0, deg1, b1r, a1r, W2)

    acc2 = _mp_kernel(srcc, dstc, g2, zacc)

    out = pl.pallas_call(
        _tc3_body,
        grid=grid,
        in_specs=[_row_spec(D), _row_spec(D), _row_spec(D),
                  _row_spec(DEGW), _row_spec(DEGW),
                  _full_spec((1, D)), _full_spec((1, D))],
        out_specs=_row_spec(D),
        out_shape=jax.ShapeDtypeStruct((N, D), jnp.float32),
    )(acc2[0], acc2[1], g2, deg0, deg1, b2r, a2r)

    return out


# speedup vs baseline: 3.3879x; 1.0431x over previous
import jax
import jax.numpy as jnp
from jax import lax
from jax.experimental import pallas as pl
from jax.experimental.pallas import tpu as pltpu
from jax.experimental.pallas import tpu_sc as plsc

N = 10000
E = 320000
D = 128

NC = 2
NS = 16
NW = NC * NS

CH = 125
CPW = 80
PAD_ROWS = NW * CPW

IGRP = 40
NGRP = CPW // IGRP
NBUF = 2

N_PAD = 10112
RPT = N_PAD // NS

DEGW = 128

_mesh = plsc.VectorSubcoreMesh(
    core_axis_name="c", subcore_axis_name="s", num_cores=NC, num_subcores=NS)


def _deg_body(dstc_hbm, zrows_hbm, ones_hbm, out_hbm, didx, ones_v, dacc):
    c = lax.axis_index("c")
    s = lax.axis_index("s")
    w = s * NC + c
    base = w * CPW

    pltpu.sync_copy(dstc_hbm.at[pl.ds(base, CPW)], didx)
    pltpu.sync_copy(ones_hbm, ones_v)
    pltpu.sync_copy(zrows_hbm, dacc.at[pl.ds(s * RPT, RPT)])
    plsc.subcore_barrier()

    def body(j, _):
        pltpu.sync_copy(ones_v, dacc.at[didx.at[j]], add=True)
        return 0
    lax.fori_loop(0, CPW, body, 0)

    plsc.subcore_barrier()
    pltpu.sync_copy(dacc.at[pl.ds(s * RPT, RPT)],
                    out_hbm.at[c, pl.ds(s * RPT, RPT)])


_deg_kernel = pl.kernel(
    _deg_body,
    out_type=jax.ShapeDtypeStruct((NC, N_PAD, DEGW), jnp.float32),
    mesh=_mesh,
    scratch_types=[
        pltpu.VMEM((CPW, CH), jnp.int32),
        pltpu.VMEM((CH, DEGW), jnp.float32),
        pltpu.VMEM_SHARED((N_PAD, DEGW), jnp.float32),
    ],
)


def _mp_body(srcc_hbm, dstc_hbm, g_hbm, zrows_hbm, out_hbm,
             sidx, didx, rows, acc, sems):
    c = lax.axis_index("c")
    s = lax.axis_index("s")
    w = s * NC + c
    base = w * CPW

    pltpu.sync_copy(zrows_hbm, acc.at[pl.ds(s * RPT, RPT)])
    plsc.subcore_barrier()

    def issue(j, b):
        pltpu.async_copy(g_hbm.at[sidx.at[j]], rows.at[b], sems.at[b])

    def group(g, _):
        gbase = base + g * IGRP
        pltpu.sync_copy(srcc_hbm.at[pl.ds(gbase, IGRP)], sidx)
        pltpu.sync_copy(dstc_hbm.at[pl.ds(gbase, IGRP)], didx)

        for b in range(NBUF - 1):
            issue(b, b)

        def body(i, _):
            j0 = i * NBUF
            for b in range(NBUF):
                j = j0 + b
                nb = (b + NBUF - 1) % NBUF
                @pl.when(j + NBUF - 1 < IGRP)
                def _():
                    issue(j + NBUF - 1, nb)
                pltpu.make_async_copy(
                    g_hbm.at[sidx.at[j]], rows.at[b], sems.at[b]).wait()
                pltpu.sync_copy(rows.at[b], acc.at[didx.at[j]], add=True)
            return 0
        lax.fori_loop(0, IGRP // NBUF, body, 0)
        return 0
    lax.fori_loop(0, NGRP, group, 0)

    plsc.subcore_barrier()
    pltpu.sync_copy(acc.at[pl.ds(s * RPT, RPT)],
                    out_hbm.at[c, pl.ds(s * RPT, RPT)])


_mp_kernel = pl.kernel(
    _mp_body,
    out_type=jax.ShapeDtypeStruct((NC, N_PAD, D), jnp.float32),
    mesh=_mesh,
    scratch_types=[
        pltpu.VMEM((IGRP, CH), jnp.int32),
        pltpu.VMEM((IGRP, CH), jnp.int32),
        pltpu.VMEM((NBUF, CH, D), jnp.float32),
        pltpu.VMEM_SHARED((N_PAD, D), jnp.float32),
        pltpu.SemaphoreType.DMA((NBUF,)),
    ],
)


BN = 1000


DVW = 8


def _tc1_body(x_ref, deg_ref, w1_ref, g1_ref, dinv_ref):
    deg = deg_ref[0, :, 0:1] + deg_ref[1, :, 0:1] + 1.0
    dinv = lax.rsqrt(jnp.maximum(deg, 1.0))
    h = jnp.dot(x_ref[...], w1_ref[...], preferred_element_type=jnp.float32)
    g1_ref[...] = h * dinv
    dinv_ref[...] = jnp.broadcast_to(dinv, (BN, DVW))


def _tc2_body(acc_ref, g1_ref, dinv_ref, b1_ref, al1_ref, w2_ref, g2_ref):
    dinv = dinv_ref[:, 0:1]
    pre = dinv * (acc_ref[0] + acc_ref[1] + g1_ref[...]) + b1_ref[...]
    z = jnp.where(pre >= 0, pre, al1_ref[...] * pre)
    h2 = jnp.dot(z, w2_ref[...], preferred_element_type=jnp.float32)
    g2_ref[...] = h2 * dinv


def _tc3_body(acc_ref, g2_ref, dinv_ref, b2_ref, al2_ref, out_ref):
    dinv = dinv_ref[:, 0:1]
    pre = dinv * (acc_ref[0] + acc_ref[1] + g2_ref[...]) + b2_ref[...]
    out_ref[...] = jnp.where(pre >= 0, pre, al2_ref[...] * pre)


def _row_spec(width):
    return pl.BlockSpec((BN, width), lambda i: (i, 0))


def _pair_spec(width):
    return pl.BlockSpec((NC, BN, width), lambda i: (0, i, 0))


def _full_spec(shape):
    return pl.BlockSpec(shape, lambda i: tuple(0 for _ in shape))


def kernel(x, edge_index, W1, b1, a1, W2, b2, a2):
    srcc = edge_index[0].reshape(PAD_ROWS, CH)
    dstc = edge_index[1].reshape(PAD_ROWS, CH)

    zdeg = jnp.zeros((RPT, DEGW), jnp.float32)
    zacc = jnp.zeros((RPT, D), jnp.float32)

    ones = jnp.ones((CH, DEGW), jnp.float32)
    degp = _deg_kernel(dstc, zdeg, ones)

    b1r = b1.reshape(1, D)
    a1r = a1.reshape(1, D)
    b2r = b2.reshape(1, D)
    a2r = a2.reshape(1, D)

    grid = (N // BN,)
    g1, dinv8 = pl.pallas_call(
        _tc1_body,
        grid=grid,
        in_specs=[_row_spec(D), _pair_spec(DEGW), _full_spec((D, D))],
        out_specs=[_row_spec(D), _row_spec(DVW)],
        out_shape=[jax.ShapeDtypeStruct((N, D), jnp.float32),
                   jax.ShapeDtypeStruct((N, DVW), jnp.float32)],
    )(x, degp, W1)

    acc1 = _mp_kernel(srcc, dstc, g1, zacc)

    g2 = pl.pallas_call(
        _tc2_body,
        grid=grid,
        in_specs=[_pair_spec(D), _row_spec(D), _row_spec(DVW),
                  _full_spec((1, D)), _full_spec((1, D)), _full_spec((D, D))],
        out_specs=_row_spec(D),
        out_shape=jax.ShapeDtypeStruct((N, D), jnp.float32),
    )(acc1, g1, dinv8, b1r, a1r, W2)

    acc2 = _mp_kernel(srcc, dstc, g2, zacc)

    out = pl.pallas_call(
        _tc3_body,
        grid=grid,
        in_specs=[_pair_spec(D), _row_spec(D), _row_spec(DVW),
                  _full_spec((1, D)), _full_spec((1, D))],
        out_specs=_row_spec(D),
        out_shape=jax.ShapeDtypeStruct((N, D), jnp.float32),
    )(acc2, g2, dinv8, b2r, a2r)

    return out


# speedup vs baseline: 3.9744x; 1.1731x over previous
import jax
import jax.numpy as jnp
from jax import lax
from jax.experimental import pallas as pl
from jax.experimental.pallas import tpu as pltpu
from jax.experimental.pallas import tpu_sc as plsc

N = 10000
E = 320000
D = 128

NC = 2
NS = 16
NW = NC * NS

CH = 125
CPW = 80
PAD_ROWS = NW * CPW

IGRP = 40
NGRP = CPW // IGRP
NBUF = 2

N_PAD = 10112
RPT = N_PAD // NS

DEGW = 128

_mesh = plsc.VectorSubcoreMesh(
    core_axis_name="c", subcore_axis_name="s", num_cores=NC, num_subcores=NS)


HR = 80
ZH = HR * 128
EPW = CPW * CH
ESTG = 2000
CT = 10
CRPT = HR // CT


def _deg_body(dst1d_hbm, zhist_hbm, out_hbm, didx1d, hist, tbuf, part, dacc16):
    c = lax.axis_index("c")
    s = lax.axis_index("s")
    w = s * NC + c
    ebase = w * EPW

    pltpu.sync_copy(zhist_hbm, hist)
    ones16 = jnp.ones((16,), jnp.float32)

    def stage(k, _):
        pltpu.sync_copy(dst1d_hbm.at[pl.ds(ebase + k * ESTG, ESTG)], didx1d)

        def cnt(i, _):
            idx = didx1d[pl.ds(i * 16, 16)]
            row = lax.shift_right_logical(idx, 7)
            col = jnp.bitwise_and(idx, 127)
            plsc.addupdate_scatter(hist, [row, col], ones16)
            return 0
        lax.fori_loop(0, ESTG // 16, cnt, 0)
        return 0
    lax.fori_loop(0, EPW // ESTG, stage, 0)

    pltpu.sync_copy(hist, dacc16.at[s])
    plsc.subcore_barrier()

    @pl.when(s < CT)
    def _():
        pltpu.sync_copy(dacc16.at[0, pl.ds(s * CRPT, CRPT)], part)

        def comb(t, _):
            pltpu.sync_copy(dacc16.at[t, pl.ds(s * CRPT, CRPT)], tbuf)

            def addr(r, _):
                def addv(i, _):
                    sl = pl.ds(i * 16, 16)
                    part[r, sl] = part[r, sl] + tbuf[r, sl]
                    return 0
                lax.fori_loop(0, 8, addv, 0)
                return 0
            lax.fori_loop(0, CRPT, addr, 0)
            return 0
        lax.fori_loop(1, NS, comb, 0)

        pltpu.sync_copy(part, out_hbm.at[c, pl.ds(s * CRPT, CRPT)])


_deg_kernel = pl.kernel(
    _deg_body,
    out_type=jax.ShapeDtypeStruct((NC, HR, 128), jnp.float32),
    mesh=_mesh,
    scratch_types=[
        pltpu.VMEM((ESTG,), jnp.int32),
        pltpu.VMEM((HR, 128), jnp.float32),
        pltpu.VMEM((CRPT, 128), jnp.float32),
        pltpu.VMEM((CRPT, 128), jnp.float32),
        pltpu.VMEM_SHARED((NS, HR, 128), jnp.float32),
    ],
    compiler_params=pltpu.CompilerParams(needs_layout_passes=False),
)


def _mp_body(srcc_hbm, dstc_hbm, g_hbm, zrows_hbm, out_hbm,
             sidx, didx, rows, acc, sems):
    c = lax.axis_index("c")
    s = lax.axis_index("s")
    w = s * NC + c
    base = w * CPW

    pltpu.sync_copy(zrows_hbm, acc.at[pl.ds(s * RPT, RPT)])
    plsc.subcore_barrier()

    def issue(j, b):
        pltpu.async_copy(g_hbm.at[sidx.at[j]], rows.at[b], sems.at[b])

    def group(g, _):
        gbase = base + g * IGRP
        pltpu.sync_copy(srcc_hbm.at[pl.ds(gbase, IGRP)], sidx)
        pltpu.sync_copy(dstc_hbm.at[pl.ds(gbase, IGRP)], didx)

        for b in range(NBUF - 1):
            issue(b, b)

        def body(i, _):
            j0 = i * NBUF
            for b in range(NBUF):
                j = j0 + b
                nb = (b + NBUF - 1) % NBUF
                @pl.when(j + NBUF - 1 < IGRP)
                def _():
                    issue(j + NBUF - 1, nb)
                pltpu.make_async_copy(
                    g_hbm.at[sidx.at[j]], rows.at[b], sems.at[b]).wait()
                pltpu.sync_copy(rows.at[b], acc.at[didx.at[j]], add=True)
            return 0
        lax.fori_loop(0, IGRP // NBUF, body, 0)
        return 0
    lax.fori_loop(0, NGRP, group, 0)

    plsc.subcore_barrier()
    pltpu.sync_copy(acc.at[pl.ds(s * RPT, RPT)],
                    out_hbm.at[c, pl.ds(s * RPT, RPT)])


_mp_kernel = pl.kernel(
    _mp_body,
    out_type=jax.ShapeDtypeStruct((NC, N_PAD, D), jnp.float32),
    mesh=_mesh,
    scratch_types=[
        pltpu.VMEM((IGRP, CH), jnp.int32),
        pltpu.VMEM((IGRP, CH), jnp.int32),
        pltpu.VMEM((NBUF, CH, D), jnp.float32),
        pltpu.VMEM_SHARED((N_PAD, D), jnp.float32),
        pltpu.SemaphoreType.DMA((NBUF,)),
    ],
)


BN = 1000


DVW = 8


def _tc1_body(x_ref, deg_ref, w1_ref, g1_ref, dinv_ref):
    deg = deg_ref[:, 0:1] + deg_ref[:, 1:2] + 1.0
    dinv = lax.rsqrt(jnp.maximum(deg, 1.0))
    h = jnp.dot(x_ref[...], w1_ref[...], preferred_element_type=jnp.float32)
    g1_ref[...] = h * dinv
    dinv_ref[...] = jnp.broadcast_to(dinv, (BN, DVW))


def _tc2_body(acc_ref, g1_ref, dinv_ref, b1_ref, al1_ref, w2_ref, g2_ref):
    dinv = dinv_ref[:, 0:1]
    pre = dinv * (acc_ref[0] + acc_ref[1] + g1_ref[...]) + b1_ref[...]
    z = jnp.where(pre >= 0, pre, al1_ref[...] * pre)
    h2 = jnp.dot(z, w2_ref[...], preferred_element_type=jnp.float32)
    g2_ref[...] = h2 * dinv


def _tc3_body(acc_ref, g2_ref, dinv_ref, b2_ref, al2_ref, out_ref):
    dinv = dinv_ref[:, 0:1]
    pre = dinv * (acc_ref[0] + acc_ref[1] + g2_ref[...]) + b2_ref[...]
    out_ref[...] = jnp.where(pre >= 0, pre, al2_ref[...] * pre)


def _row_spec(width):
    return pl.BlockSpec((BN, width), lambda i: (i, 0))


def _pair_spec(width):
    return pl.BlockSpec((NC, BN, width), lambda i: (0, i, 0))


def _full_spec(shape):
    return pl.BlockSpec(shape, lambda i: tuple(0 for _ in shape))


def kernel(x, edge_index, W1, b1, a1, W2, b2, a2):
    srcc = edge_index[0].reshape(PAD_ROWS, CH)
    dstc = edge_index[1].reshape(PAD_ROWS, CH)

    zacc = jnp.zeros((RPT, D), jnp.float32)

    zhist = jnp.zeros((HR, 128), jnp.float32)
    degp = _deg_kernel(edge_index[1], zhist)
    degT = degp.reshape(NC, ZH).T

    b1r = b1.reshape(1, D)
    a1r = a1.reshape(1, D)
    b2r = b2.reshape(1, D)
    a2r = a2.reshape(1, D)

    grid = (N // BN,)
    g1, dinv8 = pl.pallas_call(
        _tc1_body,
        grid=grid,
        in_specs=[_row_spec(D), _row_spec(2), _full_spec((D, D))],
        out_specs=[_row_spec(D), _row_spec(DVW)],
        out_shape=[jax.ShapeDtypeStruct((N, D), jnp.float32),
                   jax.ShapeDtypeStruct((N, DVW), jnp.float32)],
    )(x, degT, W1)

    acc1 = _mp_kernel(srcc, dstc, g1, zacc)

    g2 = pl.pallas_call(
        _tc2_body,
        grid=grid,
        in_specs=[_pair_spec(D), _row_spec(D), _row_spec(DVW),
                  _full_spec((1, D)), _full_spec((1, D)), _full_spec((D, D))],
        out_specs=_row_spec(D),
        out_shape=jax.ShapeDtypeStruct((N, D), jnp.float32),
    )(acc1, g1, dinv8, b1r, a1r, W2)

    acc2 = _mp_kernel(srcc, dstc, g2, zacc)

    out = pl.pallas_call(
        _tc3_body,
        grid=grid,
        in_specs=[_pair_spec(D), _row_spec(D), _row_spec(DVW),
                  _full_spec((1, D)), _full_spec((1, D))],
        out_specs=_row_spec(D),
        out_shape=jax.ShapeDtypeStruct((N, D), jnp.float32),
    )(acc2, g2, dinv8, b2r, a2r)

    return out
